# baseline (device time: 1002370 ns/iter reference)
import jax
import jax.numpy as jnp
from jax import lax
from jax.experimental import pallas as pl
from jax.experimental.pallas import tpu as pltpu

T = 2048
D = 1024
V_LOCAL = 16384


def kernel(ids, E):
    my_y = lax.axis_index("y")
    local = ids - my_y * V_LOCAL
    mask = (local >= 0) & (local < V_LOCAL)
    safe = jnp.where(mask, local, 0)
    partial = jnp.where(mask[:, None], E[safe], jnp.float32(0.0))

    def body(p_ref, out_ref, comm_ref, send_sem, recv_sem):
        my_x = lax.axis_index("x")
        my_y = lax.axis_index("y")
        peer = (my_x, 1 - my_y)

        barrier_sem = pltpu.get_barrier_semaphore()
        pl.semaphore_signal(
            barrier_sem, inc=1, device_id=peer,
            device_id_type=pl.DeviceIdType.MESH,
        )
        pl.semaphore_wait(barrier_sem, 1)

        rdma = pltpu.make_async_remote_copy(
            src_ref=p_ref,
            dst_ref=comm_ref,
            send_sem=send_sem,
            recv_sem=recv_sem,
            device_id=peer,
            device_id_type=pl.DeviceIdType.MESH,
        )
        rdma.start()
        rdma.wait()
        out_ref[...] = p_ref[...] + comm_ref[...]

    return pl.pallas_call(
        body,
        out_shape=jax.ShapeDtypeStruct((T, D), jnp.float32),
        in_specs=[pl.BlockSpec(memory_space=pltpu.VMEM)],
        out_specs=pl.BlockSpec(memory_space=pltpu.VMEM),
        scratch_shapes=[
            pltpu.VMEM((T, D), jnp.float32),
            pltpu.SemaphoreType.DMA,
            pltpu.SemaphoreType.DMA,
        ],
        compiler_params=pltpu.CompilerParams(collective_id=0),
    )(partial)


# device time: 90583 ns/iter; 11.0658x vs baseline; 11.0658x over previous
import jax
import jax.numpy as jnp
from jax import lax
from jax.experimental import pallas as pl
from jax.experimental.pallas import tpu as pltpu

T = 2048
D = 1024
V_LOCAL = 16384
T_HALF = T // 2
K = 16
CH = T_HALF // K


def kernel(ids, E):
    my_x = lax.axis_index("x")
    my_y = lax.axis_index("y")

    ids_half = lax.dynamic_slice(ids, (my_x * T_HALF,), (T_HALF,))
    local = ids_half - my_y * V_LOCAL
    valid = (local >= 0) & (local < V_LOCAL)
    safe_idx = jnp.clip(local, 0, V_LOCAL - 1)
    valid_f = valid.astype(jnp.float32)[:, None]

    def body(idx_ref, valid_ref, e_ref, out_ref, gbuf, pbuf,
             gsems, ysend, yrecv, xsend, xrecv):
        mx = lax.axis_index("x")
        my = lax.axis_index("y")
        y_peer = (mx, 1 - my)
        x_peer = (1 - mx, my)
        base = mx * T_HALF

        barrier_sem = pltpu.get_barrier_semaphore()
        for peer in (y_peer, x_peer):
            pl.semaphore_signal(
                barrier_sem, inc=1, device_id=peer,
                device_id_type=pl.DeviceIdType.MESH,
            )
        pl.semaphore_wait(barrier_sem, 2)

        def gather_row(i, _):
            pltpu.make_async_copy(
                e_ref.at[idx_ref[i]], gbuf.at[i], gsems.at[i // CH]
            ).start()
            return 0

        def wait_row(i, _):
            pltpu.make_async_copy(
                e_ref.at[idx_ref[i]], gbuf.at[i], gsems.at[i // CH]
            ).wait()
            return 0

        y_rdmas = []
        for c in range(K):
            lax.fori_loop(c * CH, (c + 1) * CH, gather_row, 0)
            lax.fori_loop(c * CH, (c + 1) * CH, wait_row, 0)
            rdma = pltpu.make_async_remote_copy(
                src_ref=gbuf.at[pl.ds(c * CH, CH)],
                dst_ref=pbuf.at[pl.ds(c * CH, CH)],
                send_sem=ysend.at[c],
                recv_sem=yrecv.at[c],
                device_id=y_peer,
                device_id_type=pl.DeviceIdType.MESH,
            )
            rdma.start()
            y_rdmas.append(rdma)

        x_rdmas = []
        for c in range(K):
            y_rdmas[c].wait_recv()
            sl = pl.ds(c * CH, CH)
            v = valid_ref[sl, :]
            o = v * gbuf[sl, :] + (1.0 - v) * pbuf[sl, :]
            osl = pl.ds(base + c * CH, CH)
            out_ref[osl, :] = o
            rdma = pltpu.make_async_remote_copy(
                src_ref=out_ref.at[osl],
                dst_ref=out_ref.at[osl],
                send_sem=xsend.at[c],
                recv_sem=xrecv.at[c],
                device_id=x_peer,
                device_id_type=pl.DeviceIdType.MESH,
            )
            rdma.start()
            x_rdmas.append(rdma)

        for c in range(K):
            y_rdmas[c].wait_send()
            x_rdmas[c].wait_send()
            x_rdmas[c].wait_recv()

    return pl.pallas_call(
        body,
        out_shape=jax.ShapeDtypeStruct((T, D), jnp.float32),
        in_specs=[
            pl.BlockSpec(memory_space=pltpu.SMEM),
            pl.BlockSpec(memory_space=pltpu.VMEM),
            pl.BlockSpec(memory_space=pl.ANY),
        ],
        out_specs=pl.BlockSpec(memory_space=pltpu.VMEM),
        scratch_shapes=[
            pltpu.VMEM((T_HALF, D), jnp.float32),
            pltpu.VMEM((T_HALF, D), jnp.float32),
            pltpu.SemaphoreType.DMA((K,)),
            pltpu.SemaphoreType.DMA((K,)),
            pltpu.SemaphoreType.DMA((K,)),
            pltpu.SemaphoreType.DMA((K,)),
            pltpu.SemaphoreType.DMA((K,)),
        ],
        compiler_params=pltpu.CompilerParams(collective_id=0),
    )(safe_idx, valid_f, E)


# device time: 89961 ns/iter; 11.1423x vs baseline; 1.0069x over previous
import jax
import jax.numpy as jnp
from jax import lax
from jax.experimental import pallas as pl
from jax.experimental.pallas import tpu as pltpu

T = 2048
D = 1024
V_LOCAL = 16384
T_HALF = T // 2
K = 16
CH = T_HALF // K


def kernel(ids, E):
    my_x = lax.axis_index("x")
    my_y = lax.axis_index("y")

    ids_half = lax.dynamic_slice(ids, (my_x * T_HALF,), (T_HALF,))
    local = ids_half - my_y * V_LOCAL
    valid = (local >= 0) & (local < V_LOCAL)
    safe_idx = jnp.clip(local, 0, V_LOCAL - 1)
    valid_f = valid.astype(jnp.float32)[:, None]

    def body(idx_ref, valid_ref, e_ref, out_ref, gbuf, pbuf,
             gsems, ysend, yrecv, xsend, xrecv):
        mx = lax.axis_index("x")
        my = lax.axis_index("y")
        y_peer = (mx, 1 - my)
        x_peer = (1 - mx, my)
        base = mx * T_HALF

        barrier_sem = pltpu.get_barrier_semaphore()
        for peer in (y_peer, x_peer):
            pl.semaphore_signal(
                barrier_sem, inc=1, device_id=peer,
                device_id_type=pl.DeviceIdType.MESH,
            )
        pl.semaphore_wait(barrier_sem, 2)

        def gather_row(i, _):
            pltpu.make_async_copy(
                e_ref.at[idx_ref[i]], gbuf.at[i], gsems.at[i // CH]
            ).start()
            return 0

        def wait_row(i, _):
            pltpu.make_async_copy(
                e_ref.at[idx_ref[i]], gbuf.at[i], gsems.at[i // CH]
            ).wait()
            return 0

        y_rdmas = []
        for c in range(K):
            lax.fori_loop(c * CH, (c + 1) * CH, gather_row, 0, unroll=8)
            lax.fori_loop(c * CH, (c + 1) * CH, wait_row, 0, unroll=8)
            rdma = pltpu.make_async_remote_copy(
                src_ref=gbuf.at[pl.ds(c * CH, CH)],
                dst_ref=pbuf.at[pl.ds(c * CH, CH)],
                send_sem=ysend.at[c],
                recv_sem=yrecv.at[c],
                device_id=y_peer,
                device_id_type=pl.DeviceIdType.MESH,
            )
            rdma.start()
            y_rdmas.append(rdma)

        x_rdmas = []
        for c in range(K):
            y_rdmas[c].wait_recv()
            sl = pl.ds(c * CH, CH)
            v = valid_ref[sl, :]
            o = v * gbuf[sl, :] + (1.0 - v) * pbuf[sl, :]
            osl = pl.ds(base + c * CH, CH)
            out_ref[osl, :] = o
            rdma = pltpu.make_async_remote_copy(
                src_ref=out_ref.at[osl],
                dst_ref=out_ref.at[osl],
                send_sem=xsend.at[c],
                recv_sem=xrecv.at[c],
                device_id=x_peer,
                device_id_type=pl.DeviceIdType.MESH,
            )
            rdma.start()
            x_rdmas.append(rdma)

        for c in range(K):
            y_rdmas[c].wait_send()
            x_rdmas[c].wait_send()
            x_rdmas[c].wait_recv()

    return pl.pallas_call(
        body,
        out_shape=jax.ShapeDtypeStruct((T, D), jnp.float32),
        in_specs=[
            pl.BlockSpec(memory_space=pltpu.SMEM),
            pl.BlockSpec(memory_space=pltpu.VMEM),
            pl.BlockSpec(memory_space=pl.ANY),
        ],
        out_specs=pl.BlockSpec(memory_space=pltpu.VMEM),
        scratch_shapes=[
            pltpu.VMEM((T_HALF, D), jnp.float32),
            pltpu.VMEM((T_HALF, D), jnp.float32),
            pltpu.SemaphoreType.DMA((K,)),
            pltpu.SemaphoreType.DMA((K,)),
            pltpu.SemaphoreType.DMA((K,)),
            pltpu.SemaphoreType.DMA((K,)),
            pltpu.SemaphoreType.DMA((K,)),
        ],
        compiler_params=pltpu.CompilerParams(collective_id=0),
    )(safe_idx, valid_f, E)


# device time: 60992 ns/iter; 16.4345x vs baseline; 1.4750x over previous
import jax
import jax.numpy as jnp
from jax import lax
from jax.experimental import pallas as pl
from jax.experimental.pallas import tpu as pltpu

T = 2048
D = 1024
V_LOCAL = 16384
T_HALF = T // 2
K = 32
CH = T_HALF // K
PIPE_LAG = 3


def kernel(ids, E):
    my_x = lax.axis_index("x")
    my_y = lax.axis_index("y")

    ids_half = lax.dynamic_slice(ids, (my_x * T_HALF,), (T_HALF,))
    local = ids_half - my_y * V_LOCAL
    valid = (local >= 0) & (local < V_LOCAL)
    safe_idx = jnp.clip(local, 0, V_LOCAL - 1)
    valid_f = jnp.broadcast_to(
        valid.astype(jnp.float32)[:, None], (T_HALF, D)
    )

    def body(idx_ref, valid_ref, e_ref, out_ref, gbuf, pbuf,
             gsems, ysend, yrecv, xsend, xrecv):
        mx = lax.axis_index("x")
        my = lax.axis_index("y")
        y_peer = (mx, 1 - my)
        x_peer = (1 - mx, my)
        base = mx * T_HALF

        barrier_sem = pltpu.get_barrier_semaphore()
        for peer in (y_peer, x_peer):
            pl.semaphore_signal(
                barrier_sem, inc=1, device_id=peer,
                device_id_type=pl.DeviceIdType.MESH,
            )
        pl.semaphore_wait(barrier_sem, 2)

        def gather_row(i, _):
            pltpu.make_async_copy(
                e_ref.at[idx_ref[i]], gbuf.at[i], gsems.at[i // CH]
            ).start()
            return 0

        def wait_row(i, _):
            pltpu.make_async_copy(
                e_ref.at[idx_ref[i]], gbuf.at[i], gsems.at[i // CH]
            ).wait()
            return 0

        def issue_chunk(c):
            lax.fori_loop(c * CH, (c + 1) * CH, gather_row, 0, unroll=8)

        def wait_chunk(c):
            lax.fori_loop(c * CH, (c + 1) * CH, wait_row, 0, unroll=8)

        y_rdmas = []
        x_rdmas = []

        def process(cc):
            y_rdmas[cc].wait_recv()
            sl = pl.ds(cc * CH, CH)
            p = pbuf[sl, :]
            o = p + valid_ref[sl, :] * (gbuf[sl, :] - p)
            osl = pl.ds(base + cc * CH, CH)
            out_ref[osl, :] = o
            rdma = pltpu.make_async_remote_copy(
                src_ref=out_ref.at[osl],
                dst_ref=out_ref.at[osl],
                send_sem=xsend.at[cc],
                recv_sem=xrecv.at[cc],
                device_id=x_peer,
                device_id_type=pl.DeviceIdType.MESH,
            )
            rdma.start()
            x_rdmas.append(rdma)

        issue_chunk(0)
        for c in range(K):
            if c + 1 < K:
                issue_chunk(c + 1)
            wait_chunk(c)
            rdma = pltpu.make_async_remote_copy(
                src_ref=gbuf.at[pl.ds(c * CH, CH)],
                dst_ref=pbuf.at[pl.ds(c * CH, CH)],
                send_sem=ysend.at[c],
                recv_sem=yrecv.at[c],
                device_id=y_peer,
                device_id_type=pl.DeviceIdType.MESH,
            )
            rdma.start()
            y_rdmas.append(rdma)
            if c >= PIPE_LAG:
                process(c - PIPE_LAG)
        for cc in range(K - PIPE_LAG, K):
            process(cc)

        for c in range(K):
            y_rdmas[c].wait_send()
            x_rdmas[c].wait_send()
            x_rdmas[c].wait_recv()

    return pl.pallas_call(
        body,
        out_shape=jax.ShapeDtypeStruct((T, D), jnp.float32),
        in_specs=[
            pl.BlockSpec(memory_space=pltpu.SMEM),
            pl.BlockSpec(memory_space=pltpu.VMEM),
            pl.BlockSpec(memory_space=pl.ANY),
        ],
        out_specs=pl.BlockSpec(memory_space=pltpu.VMEM),
        scratch_shapes=[
            pltpu.VMEM((T_HALF, D), jnp.float32),
            pltpu.VMEM((T_HALF, D), jnp.float32),
            pltpu.SemaphoreType.DMA((K,)),
            pltpu.SemaphoreType.DMA((K,)),
            pltpu.SemaphoreType.DMA((K,)),
            pltpu.SemaphoreType.DMA((K,)),
            pltpu.SemaphoreType.DMA((K,)),
        ],
        compiler_params=pltpu.CompilerParams(collective_id=0),
    )(safe_idx, valid_f, E)
